# Initial kernel scaffold; baseline (speedup 1.0000x reference)
import jax
import jax.numpy as jnp
from jax.experimental import pallas as pl


def kernel(node_features, adjacency_list, W1, b1, R1, W2, b2, R2, Wf, bf):
    raise NotImplementedError("write your pallas kernel here")


# speedup vs baseline: 3.7587x; 3.7587x over previous
import functools

import jax
import jax.numpy as jnp
from jax import lax
from jax.experimental import pallas as pl
from jax.experimental.pallas import tpu as pltpu
from jax.experimental.pallas import tpu_sc as plsc

N = 50000
D = 128
NUM_REL = 8
H1 = 64
H2 = 32
LW = 128

NSLAB = NUM_REL + 1
NW = 32
PER_W = 1568
NP = NW * PER_W
NCH = 7
CH = PER_W // NCH


def _tc_slab_body(h, x_ref, w_ref, b_ref, o_ref):
    y = jax.lax.dot_general(
        x_ref[:, :w_ref.shape[0]], w_ref[...], (((1,), (0,)), ((), ())),
        preferred_element_type=jnp.float32) + b_ref[...]
    for s in range(NSLAB):
        o_ref[s, :, 0:h] = y[:, s * h:(s + 1) * h]


def _tc_slab(x, wbig, bbig, h, blk):
    n_rows = x.shape[0]
    d_in, d_out = wbig.shape
    nb = n_rows // blk
    return pl.pallas_call(
        functools.partial(_tc_slab_body, h),
        grid=(nb,),
        in_specs=[
            pl.BlockSpec((blk, x.shape[1]), lambda i: (i, 0)),
            pl.BlockSpec((d_in, d_out), lambda i: (0, 0)),
            pl.BlockSpec((1, d_out), lambda i: (0, 0)),
        ],
        out_specs=pl.BlockSpec((NSLAB, blk, LW), lambda i: (0, i, 0)),
        out_shape=jax.ShapeDtypeStruct((NSLAB, n_rows, LW), jnp.float32),
    )(x, wbig, bbig)


def _make_sc_gather(h):
    grp = h // 16
    unr = 8
    mesh = plsc.VectorSubcoreMesh(core_axis_name="c", subcore_axis_name="s")

    @functools.partial(
        pl.kernel,
        out_type=jax.ShapeDtypeStruct((NP, LW), jnp.float32),
        mesh=mesh,
        scratch_types=[
            [pltpu.VMEM((CH,), jnp.int32) for _ in range(NSLAB)],
            pltpu.VMEM((CH, LW), jnp.float32),
            pltpu.VMEM((CH, LW), jnp.float32),
            pltpu.VMEM((CH, LW), jnp.float32),
            pltpu.SemaphoreType.DMA,
            pltpu.SemaphoreType.DMA,
            pltpu.SemaphoreType.DMA,
            pltpu.SemaphoreType.DMA,
        ],
    )
    def sc_fn(tab_hbm, idx_hbm, out_hbm, idx_v, acc_v, g0, g1, s_acc, s0, s1,
              s_idx):
        wid = lax.axis_index("s") * 2 + lax.axis_index("c")
        base0 = wid * PER_W
        bufs = (g0, g1)
        sems = (s0, s1)

        def chunk_body(c, carry):
            base = base0 + c * CH
            idx_cps = [
                pltpu.async_copy(
                    idx_hbm.at[pl.ds(s * NP + base, CH)], idx_v[s], s_idx)
                for s in range(NSLAB)
            ]
            for cp in idx_cps:
                cp.wait()
            acc_cp = pltpu.async_copy(tab_hbm.at[0].at[idx_v[0]], acc_v, s_acc)
            handles = [
                pltpu.async_copy(tab_hbm.at[1].at[idx_v[1]], g0, s0),
                pltpu.async_copy(tab_hbm.at[2].at[idx_v[2]], g1, s1),
            ]
            acc_cp.wait()
            for r in range(NUM_REL):
                p = r % 2
                buf = bufs[p]
                handles[p].wait()
                last = r == NUM_REL - 1

                def acc_body(i, _, buf=buf, last=last):
                    row = i * unr
                    for u in range(unr):
                        for j in range(grp):
                            sl = pl.ds(j * 16, 16)
                            v = acc_v[row + u, sl] + buf[row + u, sl]
                            if last:
                                v = jnp.maximum(v, 0.0)
                            acc_v[row + u, sl] = v
                    return 0

                lax.fori_loop(0, CH // unr, acc_body, 0)
                if r + 2 < NUM_REL:
                    handles[p] = pltpu.async_copy(
                        tab_hbm.at[r + 3].at[idx_v[r + 3]], buf, sems[p])
            pltpu.sync_copy(acc_v, out_hbm.at[pl.ds(base, CH)])
            return carry

        lax.fori_loop(0, NCH, chunk_body, 0)

    return sc_fn


_sc_gather_h1 = _make_sc_gather(H1)
_sc_gather_h2 = _make_sc_gather(H2)


def _tc_final_body(h2_ref, wf_ref, bf_ref, o_ref, acc_ref):
    i = pl.program_id(0)

    @pl.when(i == 0)
    def _():
        acc_ref[...] = jnp.zeros_like(acc_ref)

    acc_ref[...] += jnp.sum(h2_ref[:, 0:H2], axis=0, keepdims=True)

    @pl.when(i == pl.num_programs(0) - 1)
    def _():
        mean = acc_ref[...] * (1.0 / N)
        o_ref[...] = jax.lax.dot_general(
            mean, wf_ref[...], (((1,), (1,)), ((), ())),
            preferred_element_type=jnp.float32) + bf_ref[...]


def _tc_final(h2, wf, bf):
    blk = 1000
    return pl.pallas_call(
        _tc_final_body,
        grid=(N // blk,),
        in_specs=[
            pl.BlockSpec((blk, LW), lambda i: (i, 0)),
            pl.BlockSpec((D, H2), lambda i: (0, 0)),
            pl.BlockSpec((1, D), lambda i: (0, 0)),
        ],
        out_specs=pl.BlockSpec((1, D), lambda i: (0, 0)),
        out_shape=jax.ShapeDtypeStruct((1, D), jnp.float32),
        scratch_shapes=[pltpu.VMEM((1, H2), jnp.float32)],
    )(h2, wf, bf[None])


def kernel(node_features, adjacency_list, W1, b1, R1, W2, b2, R2, Wf, bf):
    pad = NP - N
    feats = jnp.pad(node_features, ((0, pad), (0, 0)))
    adjp = jnp.pad(adjacency_list, ((0, 0), (0, pad)))
    self_idx = jnp.arange(NP, dtype=jnp.int32)[None]
    idx = jnp.concatenate([self_idx, adjp], axis=0).reshape(NSLAB * NP)

    wbig1 = jnp.concatenate([W1.T] + [R1[r].T for r in range(NUM_REL)], axis=1)
    bbig1 = jnp.concatenate(
        [b1, jnp.zeros((NUM_REL * H1,), jnp.float32)])[None]
    wbig2 = jnp.concatenate([W2.T] + [R2[r].T for r in range(NUM_REL)], axis=1)
    bbig2 = jnp.concatenate(
        [b2, jnp.zeros((NUM_REL * H2,), jnp.float32)])[None]

    s1 = _tc_slab(feats, wbig1, bbig1, H1, blk=1792)
    h1 = _sc_gather_h1(s1, idx)
    s2 = _tc_slab(h1, wbig2, bbig2, H2, blk=1792)
    h2 = _sc_gather_h2(s2, idx)
    out = _tc_final(h2, Wf, bf)
    return out[0]


# speedup vs baseline: 4.4092x; 1.1731x over previous
import functools

import jax
import jax.numpy as jnp
from jax import lax
from jax.experimental import pallas as pl
from jax.experimental.pallas import tpu as pltpu
from jax.experimental.pallas import tpu_sc as plsc

N = 50000
D = 128
NUM_REL = 8
H1 = 64
H2 = 32
LW = 128

NSLAB = NUM_REL + 1
NW = 32
PER_W = 1568
NP = NW * PER_W
NCH = 7
CH = PER_W // NCH


def _tc_slab_body(h, x_ref, w_ref, b_ref, o_ref):
    y = jax.lax.dot_general(
        x_ref[:, :w_ref.shape[0]], w_ref[...],
        (((1,), (0,)), ((), ())),
        preferred_element_type=jnp.float32) + b_ref[...]
    for s in range(NSLAB):
        o_ref[s, :, 0:h] = y[:, s * h:(s + 1) * h]


def _tc_slab(x, wbig, bbig, h, blk):
    n_rows = x.shape[0]
    d_in, d_out = wbig.shape
    nb = n_rows // blk
    return pl.pallas_call(
        functools.partial(_tc_slab_body, h),
        grid=(nb,),
        in_specs=[
            pl.BlockSpec((blk, x.shape[1]), lambda i: (i, 0)),
            pl.BlockSpec((d_in, d_out), lambda i: (0, 0)),
            pl.BlockSpec((1, d_out), lambda i: (0, 0)),
        ],
        out_specs=pl.BlockSpec((NSLAB, blk, LW), lambda i: (0, i, 0)),
        out_shape=jax.ShapeDtypeStruct((NSLAB, n_rows, LW), jnp.float32),
    )(x, wbig, bbig)


def _make_sc_gather(h):
    grp = h // 16
    unr = 8
    mesh = plsc.VectorSubcoreMesh(core_axis_name="c", subcore_axis_name="s")

    @functools.partial(
        pl.kernel,
        out_type=jax.ShapeDtypeStruct((NP, LW), jnp.float32),
        mesh=mesh,
        scratch_types=[
            [pltpu.VMEM((CH,), jnp.int32) for _ in range(NUM_REL)],
            pltpu.VMEM((CH, LW), jnp.float32),
            pltpu.VMEM((CH, LW), jnp.float32),
            pltpu.VMEM((CH, LW), jnp.float32),
            pltpu.SemaphoreType.DMA,
            pltpu.SemaphoreType.DMA,
            pltpu.SemaphoreType.DMA,
            pltpu.SemaphoreType.DMA,
        ],
    )
    def sc_fn(tab_hbm, idx_hbm, out_hbm, idx_v, acc_v, g0, g1, s_acc, s0, s1,
              s_idx):
        wid = lax.axis_index("s") * 2 + lax.axis_index("c")
        base0 = wid * PER_W
        bufs = (g0, g1)
        sems = (s0, s1)

        def chunk_body(c, carry):
            base = base0 + c * CH
            idx_cps = [
                pltpu.async_copy(
                    idx_hbm.at[pl.ds(r * NP + base, CH)], idx_v[r], s_idx)
                for r in range(NUM_REL)
            ]
            for cp in idx_cps:
                cp.wait()
            acc_cp = pltpu.async_copy(
                tab_hbm.at[0].at[pl.ds(base, CH)], acc_v, s_acc)
            handles = [
                pltpu.async_copy(tab_hbm.at[1].at[idx_v[0]], g0, s0),
                pltpu.async_copy(tab_hbm.at[2].at[idx_v[1]], g1, s1),
            ]
            acc_cp.wait()
            for r in range(NUM_REL):
                p = r % 2
                buf = bufs[p]
                handles[p].wait()
                last = r == NUM_REL - 1

                def acc_body(i, _, buf=buf, last=last):
                    row = i * unr
                    for u in range(unr):
                        for j in range(grp):
                            sl = pl.ds(j * 16, 16)
                            v = acc_v[row + u, sl] + buf[row + u, sl]
                            if last:
                                v = jnp.maximum(v, 0.0)
                            acc_v[row + u, sl] = v
                    return 0

                lax.fori_loop(0, CH // unr, acc_body, 0)
                if r + 2 < NUM_REL:
                    handles[p] = pltpu.async_copy(
                        tab_hbm.at[r + 3].at[idx_v[r + 2]], buf, sems[p])
            pltpu.sync_copy(acc_v, out_hbm.at[pl.ds(base, CH)])
            return carry

        lax.fori_loop(0, NCH, chunk_body, 0)

    return sc_fn


_sc_gather_h1 = _make_sc_gather(H1)
_sc_gather_h2 = _make_sc_gather(H2)


def _tc_final_body(h2_ref, wf_ref, bf_ref, o_ref, acc_ref):
    i = pl.program_id(0)

    @pl.when(i == 0)
    def _():
        acc_ref[...] = jnp.zeros_like(acc_ref)

    acc_ref[...] += jnp.sum(h2_ref[:, 0:H2], axis=0, keepdims=True)

    @pl.when(i == pl.num_programs(0) - 1)
    def _():
        mean = acc_ref[...] * (1.0 / N)
        o_ref[...] = jax.lax.dot_general(
            mean, wf_ref[...], (((1,), (1,)), ((), ())),
            preferred_element_type=jnp.float32) + bf_ref[...]


def _tc_final(h2, wf, bf):
    blk = 2000
    return pl.pallas_call(
        _tc_final_body,
        grid=(N // blk,),
        in_specs=[
            pl.BlockSpec((blk, LW), lambda i: (i, 0)),
            pl.BlockSpec((D, H2), lambda i: (0, 0)),
            pl.BlockSpec((1, D), lambda i: (0, 0)),
        ],
        out_specs=pl.BlockSpec((1, D), lambda i: (0, 0)),
        out_shape=jax.ShapeDtypeStruct((1, D), jnp.float32),
        scratch_shapes=[pltpu.VMEM((1, H2), jnp.float32)],
    )(h2, wf, bf[None])


def kernel(node_features, adjacency_list, W1, b1, R1, W2, b2, R2, Wf, bf):
    pad = NP - N
    feats = jnp.pad(node_features, ((0, pad), (0, 0)))
    pad_idx = jnp.broadcast_to(
        jnp.arange(N, NP, dtype=jnp.int32), (NUM_REL, pad))
    adjp = jnp.concatenate([adjacency_list, pad_idx], axis=1)
    idx = adjp.reshape(NUM_REL * NP)

    wbig1 = jnp.concatenate([W1.T] + [R1[r].T for r in range(NUM_REL)], axis=1)
    bbig1 = jnp.concatenate(
        [b1, jnp.zeros((NUM_REL * H1,), jnp.float32)])[None]
    wbig2 = jnp.concatenate([W2.T] + [R2[r].T for r in range(NUM_REL)], axis=1)
    bbig2 = jnp.concatenate(
        [b2, jnp.zeros((NUM_REL * H2,), jnp.float32)])[None]

    s1 = _tc_slab(feats, wbig1, bbig1, H1, blk=1792)
    h1 = _sc_gather_h1(s1, idx)
    s2 = _tc_slab(h1, wbig2, bbig2, H2, blk=1792)
    h2 = _sc_gather_h2(s2, idx)
    out = _tc_final(h2, Wf, bf)
    return out[0]


# speedup vs baseline: 5.0657x; 1.1489x over previous
import functools

import jax
import jax.numpy as jnp
from jax import lax
from jax.experimental import pallas as pl
from jax.experimental.pallas import tpu as pltpu
from jax.experimental.pallas import tpu_sc as plsc

N = 50000
D = 128
NUM_REL = 8
H1 = 64
H2 = 32
LW = 128

NSLAB = NUM_REL + 1
NW = 32
PER_W = 1568
NP = NW * PER_W
NCH = 7
CH = PER_W // NCH


def _tc_slab_body(h, x_ref, w_ref, b_ref, o_ref):
    k = LW // h
    y = jax.lax.dot_general(
        x_ref[...], w_ref[...],
        (((1,), (0,)), ((), ())),
        preferred_element_type=jnp.float32) + b_ref[...]
    for s in range(NSLAB):
        p, o = divmod(s, k)
        o_ref[p, :, o * h:(o + 1) * h] = y[:, s * h:(s + 1) * h]


def _tc_slab(x, wbig, bbig, h, blk):
    n_rows = x.shape[0]
    d_in, d_out = wbig.shape
    npl = -(-NSLAB // (LW // h))
    nb = n_rows // blk
    return pl.pallas_call(
        functools.partial(_tc_slab_body, h),
        grid=(nb,),
        in_specs=[
            pl.BlockSpec((blk, d_in), lambda i: (i, 0)),
            pl.BlockSpec((d_in, d_out), lambda i: (0, 0)),
            pl.BlockSpec((1, d_out), lambda i: (0, 0)),
        ],
        out_specs=pl.BlockSpec((npl, blk, LW), lambda i: (0, i, 0)),
        out_shape=jax.ShapeDtypeStruct((npl, n_rows, LW), jnp.float32),
    )(x, wbig, bbig)


def _make_sc_gather(h):
    grp = h // 16
    k = LW // h
    unr = 8
    mesh = plsc.VectorSubcoreMesh(core_axis_name="c", subcore_axis_name="s")

    @functools.partial(
        pl.kernel,
        out_type=jax.ShapeDtypeStruct((NP, h), jnp.float32),
        mesh=mesh,
        scratch_types=[
            [pltpu.VMEM((CH,), jnp.int32) for _ in range(NUM_REL)],
            pltpu.VMEM((CH, LW), jnp.float32),
            pltpu.VMEM((CH, LW), jnp.float32),
            pltpu.VMEM((CH, LW), jnp.float32),
            pltpu.VMEM((CH, h), jnp.float32),
            pltpu.SemaphoreType.DMA,
            pltpu.SemaphoreType.DMA,
            pltpu.SemaphoreType.DMA,
            pltpu.SemaphoreType.DMA,
        ],
    )
    def sc_fn(tab_hbm, idx_hbm, out_hbm, idx_v, acc_v, g0, g1, out_v,
              s_acc, s0, s1, s_idx):
        wid = lax.axis_index("s") * 2 + lax.axis_index("c")
        base0 = wid * PER_W
        bufs = (g0, g1)
        sems = (s0, s1)

        def plane(r):
            return (r + 1) // k

        def chunk_body(c, carry):
            base = base0 + c * CH
            idx_cps = [
                pltpu.async_copy(
                    idx_hbm.at[pl.ds(r * NP + base, CH)], idx_v[r], s_idx)
                for r in range(NUM_REL)
            ]
            for cp in idx_cps:
                cp.wait()
            acc_cp = pltpu.async_copy(
                tab_hbm.at[0].at[pl.ds(base, CH)], acc_v, s_acc)
            handles = [
                pltpu.async_copy(tab_hbm.at[plane(0)].at[idx_v[0]], g0, s0),
                pltpu.async_copy(tab_hbm.at[plane(1)].at[idx_v[1]], g1, s1),
            ]
            acc_cp.wait()
            for r in range(NUM_REL):
                p = r % 2
                buf = bufs[p]
                handles[p].wait()
                last = r == NUM_REL - 1
                off = ((r + 1) % k) * h

                def acc_body(i, _, buf=buf, last=last, off=off):
                    row = i * unr
                    for u in range(unr):
                        for j in range(grp):
                            dst = pl.ds(j * 16, 16)
                            src = pl.ds(off + j * 16, 16)
                            v = acc_v[row + u, dst] + buf[row + u, src]
                            if last:
                                out_v[row + u, dst] = jnp.maximum(v, 0.0)
                            else:
                                acc_v[row + u, dst] = v
                    return 0

                lax.fori_loop(0, CH // unr, acc_body, 0)
                if r + 2 < NUM_REL:
                    handles[p] = pltpu.async_copy(
                        tab_hbm.at[plane(r + 2)].at[idx_v[r + 2]], buf,
                        sems[p])
            pltpu.sync_copy(out_v, out_hbm.at[pl.ds(base, CH)])
            return carry

        lax.fori_loop(0, NCH, chunk_body, 0)

    return sc_fn


_sc_gather_h1 = _make_sc_gather(H1)
_sc_gather_h2 = _make_sc_gather(H2)


def _tc_final_body(h2_ref, wf_ref, bf_ref, o_ref, acc_ref):
    i = pl.program_id(0)

    @pl.when(i == 0)
    def _():
        acc_ref[...] = jnp.zeros_like(acc_ref)

    acc_ref[...] += jnp.sum(h2_ref[...], axis=0, keepdims=True)

    @pl.when(i == pl.num_programs(0) - 1)
    def _():
        mean = acc_ref[...] * (1.0 / N)
        o_ref[...] = jax.lax.dot_general(
            mean, wf_ref[...], (((1,), (1,)), ((), ())),
            preferred_element_type=jnp.float32) + bf_ref[...]


def _tc_final(h2, wf, bf):
    blk = 2000
    return pl.pallas_call(
        _tc_final_body,
        grid=(N // blk,),
        in_specs=[
            pl.BlockSpec((blk, H2), lambda i: (i, 0)),
            pl.BlockSpec((D, H2), lambda i: (0, 0)),
            pl.BlockSpec((1, D), lambda i: (0, 0)),
        ],
        out_specs=pl.BlockSpec((1, D), lambda i: (0, 0)),
        out_shape=jax.ShapeDtypeStruct((1, D), jnp.float32),
        scratch_shapes=[pltpu.VMEM((1, H2), jnp.float32)],
    )(h2, wf, bf[None])


def kernel(node_features, adjacency_list, W1, b1, R1, W2, b2, R2, Wf, bf):
    pad = NP - N
    feats = jnp.pad(node_features, ((0, pad), (0, 0)))
    pad_idx = jnp.broadcast_to(
        jnp.arange(N, NP, dtype=jnp.int32), (NUM_REL, pad))
    adjp = jnp.concatenate([adjacency_list, pad_idx], axis=1)
    idx = adjp.reshape(NUM_REL * NP)

    wbig1 = jnp.concatenate([W1.T] + [R1[r].T for r in range(NUM_REL)], axis=1)
    bbig1 = jnp.concatenate(
        [b1, jnp.zeros((NUM_REL * H1,), jnp.float32)])[None]
    wbig2 = jnp.concatenate([W2.T] + [R2[r].T for r in range(NUM_REL)], axis=1)
    bbig2 = jnp.concatenate(
        [b2, jnp.zeros((NUM_REL * H2,), jnp.float32)])[None]

    s1 = _tc_slab(feats, wbig1, bbig1, H1, blk=1792)
    h1 = _sc_gather_h1(s1, idx)
    s2 = _tc_slab(h1, wbig2, bbig2, H2, blk=1792)
    h2 = _sc_gather_h2(s2, idx)
    out = _tc_final(h2, Wf, bf)
    return out[0]
